# baseline (speedup 1.0000x reference)
import functools

import jax
import jax.numpy as jnp
from jax import lax
from jax.experimental import pallas as pl
from jax.experimental.pallas import tpu as pltpu

NUM_PROMPTS = 1000
SEQ_LEN = 77
TEXT_DIM = 4096
BATCH = 256

WINDOW = 16


def _body(idx_ref, tab_ref, out_ref, sem):
    def copy(b):
        return pltpu.make_async_copy(
            tab_ref.at[idx_ref[b]], out_ref.at[b], sem
        )

    def fire(b, carry):
        copy(b).start()
        return carry

    def drain(b, carry):
        copy(b).wait()
        return carry

    lax.fori_loop(0, WINDOW, fire, 0)

    def step(b, carry):
        copy(b).wait()
        copy(b + WINDOW).start()
        return carry

    lax.fori_loop(0, BATCH - WINDOW, step, 0)
    lax.fori_loop(BATCH - WINDOW, BATCH, drain, 0)


_grid_spec = pltpu.PrefetchScalarGridSpec(
    num_scalar_prefetch=1,
    grid=(1,),
    in_specs=[pl.BlockSpec(memory_space=pl.ANY)],
    out_specs=pl.BlockSpec(memory_space=pl.ANY),
    scratch_shapes=[pltpu.SemaphoreType.DMA],
)

_gather = pl.pallas_call(
    _body,
    grid_spec=_grid_spec,
    out_shape=jax.ShapeDtypeStruct((BATCH, SEQ_LEN, TEXT_DIM), jnp.float32),
)


def kernel(prompt_idx, embeddings):
    return _gather(prompt_idx.astype(jnp.int32), embeddings)


# speedup vs baseline: 7.2401x; 7.2401x over previous
import jax
import jax.numpy as jnp
from jax.experimental import pallas as pl
from jax.experimental.pallas import tpu as pltpu

NUM_PROMPTS = 1000
SEQ_LEN = 77
TEXT_DIM = 4096
BATCH = 256


def _body(idx_ref, row_ref, out_ref):
    out_ref[...] = row_ref[...]


_grid_spec = pltpu.PrefetchScalarGridSpec(
    num_scalar_prefetch=1,
    grid=(BATCH,),
    in_specs=[
        pl.BlockSpec((1, SEQ_LEN, TEXT_DIM), lambda i, idx_ref: (idx_ref[i], 0, 0)),
    ],
    out_specs=pl.BlockSpec((1, SEQ_LEN, TEXT_DIM), lambda i, idx_ref: (i, 0, 0)),
)

_gather = pl.pallas_call(
    _body,
    grid_spec=_grid_spec,
    out_shape=jax.ShapeDtypeStruct((BATCH, SEQ_LEN, TEXT_DIM), jnp.float32),
)


def kernel(prompt_idx, embeddings):
    return _gather(prompt_idx.astype(jnp.int32), embeddings)


# speedup vs baseline: 7.5832x; 1.0474x over previous
import jax
import jax.numpy as jnp
from jax import lax
from jax.experimental import pallas as pl
from jax.experimental.pallas import tpu as pltpu

NUM_PROMPTS = 1000
SEQ_LEN = 77
TEXT_DIM = 4096
BATCH = 256

K = 8
D = 6


def _body(idx_ref, tab_ref, out_ref, buf, sem_in, sem_out):
    def slot(b):
        return lax.rem(b, K)

    def start_in(b):
        pltpu.make_async_copy(
            tab_ref.at[idx_ref[b]], buf.at[slot(b)], sem_in.at[slot(b)]
        ).start()

    def wait_in(b):
        pltpu.make_async_copy(
            tab_ref.at[idx_ref[b]], buf.at[slot(b)], sem_in.at[slot(b)]
        ).wait()

    def start_out(b):
        pltpu.make_async_copy(
            buf.at[slot(b)], out_ref.at[b], sem_out.at[slot(b)]
        ).start()

    def wait_out(b):
        pltpu.make_async_copy(
            buf.at[slot(b)], out_ref.at[b], sem_out.at[slot(b)]
        ).wait()

    for j in range(D):
        start_in(jnp.int32(j))

    def step(b, carry):
        wait_in(b)
        start_out(b)
        p = b + D

        @pl.when(p < BATCH)
        def _prefetch():
            @pl.when(p >= K)
            def _free():
                wait_out(p - K)
            start_in(p)

        return carry

    lax.fori_loop(0, BATCH, step, 0)

    for j in range(BATCH - K, BATCH):
        wait_out(jnp.int32(j))


_grid_spec = pltpu.PrefetchScalarGridSpec(
    num_scalar_prefetch=1,
    grid=(1,),
    in_specs=[pl.BlockSpec(memory_space=pl.ANY)],
    out_specs=pl.BlockSpec(memory_space=pl.ANY),
    scratch_shapes=[
        pltpu.VMEM((K, SEQ_LEN, TEXT_DIM), jnp.float32),
        pltpu.SemaphoreType.DMA((K,)),
        pltpu.SemaphoreType.DMA((K,)),
    ],
)

_gather = pl.pallas_call(
    _body,
    grid_spec=_grid_spec,
    out_shape=jax.ShapeDtypeStruct((BATCH, SEQ_LEN, TEXT_DIM), jnp.float32),
)


def kernel(prompt_idx, embeddings):
    return _gather(prompt_idx.astype(jnp.int32), embeddings)
